# baseline (speedup 1.0000x reference)
import jax
import jax.numpy as jnp
from jax import lax
from jax.experimental import pallas as pl
from jax.experimental.pallas import tpu as pltpu
from jax.experimental.pallas import tpu_sc as plsc
from jax._src.pallas import mpmd

NUM_POSITIONS = 8192
DIM = 1024
BATCH = 4
SEQ_LEN = 8192

ROWS = BATCH * SEQ_LEN
NC, NS, L = 2, 16, 16
HALF = ROWS // NC
CHUNK = 8
TSTEP = NS * CHUNK
NSTEPS = HALF // TSTEP
NBUF = 4
SNBUF = 2
GL = 2
assert NSTEPS % NBUF == 0 and NSTEPS % SNBUF == 0


def _make_kernel():
    scs_mesh = plsc.ScalarSubcoreMesh(axis_name="c", num_cores=NC)
    tec_mesh = plsc.VectorSubcoreMesh(core_axis_name="c", subcore_axis_name="s")

    tec_vmem = pltpu.VMEM @ tec_mesh
    row_buf = tec_vmem((CHUNK, DIM), jnp.float32)
    scratch_types = (
        [pltpu.VMEM_SHARED((SNBUF, TSTEP, DIM), jnp.float32)]
        + [row_buf] * (3 * NBUF)
        + [tec_vmem((CHUNK,), jnp.int32)] * NBUF
        + [pltpu.SemaphoreType.REGULAR @ tec_mesh] * SNBUF
        + [pltpu.SemaphoreType.REGULAR @ scs_mesh] * SNBUF
        + [pltpu.SemaphoreType.DMA @ scs_mesh]
        + [pltpu.SemaphoreType.DMA @ tec_mesh] * (4 * NBUF)
    )

    def split_scratch(scratch):
        spm = scratch[0]
        bufs = scratch[1:1 + 4 * NBUF]
        xv = list(bufs[0:NBUF])
        gb = list(bufs[NBUF:2 * NBUF])
        ob = list(bufs[2 * NBUF:3 * NBUF])
        ib = list(bufs[3 * NBUF:4 * NBUF])
        rest = scratch[1 + 4 * NBUF:]
        ready = list(rest[0:SNBUF])
        free = list(rest[SNBUF:2 * SNBUF])
        sdma = rest[2 * SNBUF]
        tsem = rest[2 * SNBUF + 1:]
        sxv = list(tsem[0:NBUF])
        sg = list(tsem[NBUF:2 * NBUF])
        so = list(tsem[2 * NBUF:3 * NBUF])
        sidx = list(tsem[3 * NBUF:4 * NBUF])
        return spm, xv, gb, ob, ib, ready, free, sdma, sxv, sg, so, sidx

    def scs_fn(x_hbm, pos_hbm, table_hbm, out_hbm, *scratch):
        spm, xv, gb, ob, ib, ready, free, sdma, sxv, sg, so, sidx = (
            split_scratch(scratch))
        cbase = lax.axis_index("c") * HALF

        def step_copy(g, s):
            return pltpu.make_async_copy(
                x_hbm.at[pl.ds(cbase + g * TSTEP, TSTEP)], spm.at[s], sdma)

        def signal_all(s):
            for t in range(NS):
                pl.semaphore_signal(ready[s], 1, device_id={"s": t})

        for s in range(SNBUF):
            step_copy(s, s).start()
            step_copy(s, s).wait()
            signal_all(s)

        def group(q, carry):
            for s in range(SNBUF):
                g = SNBUF + q * SNBUF + s
                pl.semaphore_wait(free[s], NS)
                step_copy(g, s).start()
                step_copy(g, s).wait()
                signal_all(s)
            return carry

        lax.fori_loop(0, (NSTEPS - SNBUF) // SNBUF, group, 0)

        for s in range(SNBUF):
            pl.semaphore_wait(free[s], NS)

    def tec_fn(x_hbm, pos_hbm, table_hbm, out_hbm, *scratch):
        spm, xv, gb, ob, ib, ready, free, sdma, sxv, sg, so, sidx = (
            split_scratch(scratch))
        cbase = lax.axis_index("c") * HALF
        sid = lax.axis_index("s")

        def row0(i):
            return cbase + i * TSTEP + sid * CHUNK

        def idx_copy(i, b):
            return pltpu.make_async_copy(
                pos_hbm.at[pl.ds(row0(i), CHUNK)], ib[b], sidx[b])

        def g_copy(b):
            return pltpu.make_async_copy(
                table_hbm.at[ib[b]], gb[b], sg[b])

        def hop(s2, b):
            return pltpu.make_async_copy(
                spm.at[s2, pl.ds(sid * CHUNK, CHUNK)], xv[b], sxv[b])

        def o_copy(i, b):
            return pltpu.make_async_copy(
                ob[b], out_hbm.at[pl.ds(row0(i), CHUNK)], so[b])

        for i in range(NBUF):
            idx_copy(i, i).start()
        for i in range(GL):
            idx_copy(i, i).wait()
            g_copy(i).start()
        pl.semaphore_wait(ready[0], 1)
        hop(0, 0).start()

        def group(q, carry):
            for b in range(NBUF):
                i = q * NBUF + b
                s2 = b % SNBUF
                sn2 = (b + 1) % SNBUF
                bn = (b + 1) % NBUF

                hop(s2, b).wait()
                pl.semaphore_signal(free[s2], 1)

                @pl.when(i + 1 < NSTEPS)
                def _():
                    pl.semaphore_wait(ready[sn2], 1)
                    hop(sn2, bn).start()

                g_copy(b).wait()

                @pl.when(i >= NBUF)
                def _():
                    o_copy(i - NBUF, b).wait()

                def add_row(r, c2):
                    for c in range(DIM // L):
                        sl = pl.ds(c * L, L)
                        ob[b][r, sl] = xv[b][r, sl] + gb[b][r, sl]
                    return c2

                lax.fori_loop(0, CHUNK, add_row, 0)
                o_copy(i, b).start()

                @pl.when(i + GL < NSTEPS)
                def _():
                    bg = (b + GL) % NBUF
                    idx_copy(i + GL, bg).wait()
                    g_copy(bg).start()

                @pl.when(i + NBUF < NSTEPS)
                def _():
                    idx_copy(i + NBUF, b).start()
            return carry

        lax.fori_loop(0, NSTEPS // NBUF, group, 0)

        for i in range(NSTEPS - NBUF, NSTEPS):
            o_copy(i, i % NBUF).wait()

    fn = mpmd.mpmd_map(
        [(scs_mesh, scs_fn), (tec_mesh, tec_fn)],
        out_types=jax.ShapeDtypeStruct((ROWS, DIM), jnp.float32),
        scratch_types=scratch_types,
    )
    return fn


_sc_kernel = _make_kernel()


@jax.jit
def kernel(x, positions, table):
    B, S, D = x.shape
    xf = x.reshape(B * S, D)
    pf = positions.reshape(B * S).astype(jnp.int32)
    out = _sc_kernel(xf, pf, table)
    return out.reshape(B, S, D)
